# baseline (speedup 1.0000x reference)
import functools
import jax
import jax.numpy as jnp
from jax import lax
from jax.experimental import pallas as pl
from jax.experimental.pallas import tpu as pltpu
from jax.experimental.pallas import tpu_sc as plsc

_B, _N = 4, 1024
_TD = 256
_RN = 128
_DC = 128
_NG = _DC // 16


def _sc_colmax_body(v5_hbm, x6_hbm, buf, out_v, sem0, sem1):
    wid = lax.axis_index("s") * 2 + lax.axis_index("c")
    b = wid // 8
    d0 = (wid % 8) * _DC
    sems = (sem0, sem1)
    nchunks = _N // _RN
    cps = [None, None]
    cps[0] = pltpu.async_copy(
        v5_hbm.at[b, pl.ds(0, _RN), pl.ds(d0, _DC)], buf.at[0], sem0)
    accs = tuple(jnp.full((16,), -jnp.inf, jnp.float32) for _ in range(_NG))
    for k in range(nchunks):
        slot = k % 2
        if k + 1 < nchunks:
            nslot = (k + 1) % 2
            cps[nslot] = pltpu.async_copy(
                v5_hbm.at[b, pl.ds((k + 1) * _RN, _RN), pl.ds(d0, _DC)],
                buf.at[nslot], sems[nslot])
        cps[slot].wait()

        def row_body(r, a, slot=slot):
            return tuple(
                jnp.maximum(a[j], buf[slot, r, pl.ds(j * 16, 16)])
                for j in range(_NG))

        accs = lax.fori_loop(0, _RN, row_body, accs)
    for j in range(_NG):
        out_v[pl.ds(j * 16, 16)] = accs[j]
    pltpu.sync_copy(out_v, x6_hbm.at[b, pl.ds(d0, _DC)])


def _sc_colmax(v5):
    B, N, D = v5.shape
    mesh = plsc.VectorSubcoreMesh(core_axis_name="c", subcore_axis_name="s")
    k = functools.partial(
        pl.kernel,
        mesh=mesh,
        out_type=jax.ShapeDtypeStruct((B, D), jnp.float32),
        scratch_types=[
            pltpu.VMEM((2, _RN, _DC), jnp.float32),
            pltpu.VMEM((_DC,), jnp.float32),
            pltpu.SemaphoreType.DMA,
            pltpu.SemaphoreType.DMA,
        ],
    )(_sc_colmax_body)
    return k(v5)


def _main_body(x1_ref, v1_ref, v7r_ref, x6_ref, x10_ref, x11_ref, x12_ref):
    x1t = x1_ref[0]
    v1t = v1_ref[0]
    v7t = v7r_ref[0]
    x6v = x6_ref[0, 0]
    x9 = jax.nn.sigmoid(x1t + v7t)
    p = x9 * v1t
    top_a = p * x1t
    x6col = x6v[:, None]
    top_b = p * x6col
    x6b = jnp.broadcast_to(x6col, x1t.shape)
    x10_ref[0, :_N, :] = x1t
    x10_ref[0, _N:, :] = x6b
    x12_ref[0, :_N, :] = x1t + top_a
    x12_ref[0, _N:, :] = x6b + top_b
    x11_ref[0, :, :_N] = top_a.T
    x11_ref[0, :, _N:] = top_b.T


def kernel(x1, v1, v5, v6r, v7r):
    del v6r
    B, N, D = x1.shape

    x6 = _sc_colmax(v5).reshape(B, 1, D)

    strip = pl.BlockSpec((1, N, _TD), lambda b, d: (b, 0, d))
    x10, x11, x12 = pl.pallas_call(
        _main_body,
        grid=(B, D // _TD),
        in_specs=[
            strip,
            strip,
            strip,
            pl.BlockSpec((1, 1, N), lambda b, d: (b, 0, 0)),
        ],
        out_specs=[
            pl.BlockSpec((1, 2 * N, _TD), lambda b, d: (b, 0, d)),
            pl.BlockSpec((1, _TD, 2 * N), lambda b, d: (b, d, 0)),
            pl.BlockSpec((1, 2 * N, _TD), lambda b, d: (b, 0, d)),
        ],
        out_shape=[
            jax.ShapeDtypeStruct((B, 2 * N, D), jnp.float32),
            jax.ShapeDtypeStruct((B, D, 2 * N), jnp.float32),
            jax.ShapeDtypeStruct((B, 2 * N, D), jnp.float32),
        ],
    )(x1, v1, v7r, x6)
    return (x10, x11, x12)


# speedup vs baseline: 1.4841x; 1.4841x over previous
import jax
import jax.numpy as jnp
from jax.experimental import pallas as pl

_B, _N = 4, 1024
_TD = 512


def _colmax_body(v5_ref, x6_ref):
    x6_ref[0, 0, :] = jnp.max(v5_ref[0], axis=0)


def _main_body(x1_ref, v1_ref, v7r_ref, x6_ref, x10_ref, x11_ref, x12_ref):
    x1t = x1_ref[0]
    v1t = v1_ref[0]
    v7t = v7r_ref[0]
    x6v = x6_ref[0, 0]
    x9 = jax.nn.sigmoid(x1t + v7t)
    p = x9 * v1t
    top_a = p * x1t
    x6col = x6v[:, None]
    top_b = p * x6col
    x6b = jnp.broadcast_to(x6col, x1t.shape)
    x10_ref[0, :_N, :] = x1t
    x10_ref[0, _N:, :] = x6b
    x12_ref[0, :_N, :] = x1t + top_a
    x12_ref[0, _N:, :] = x6b + top_b
    x11_ref[0, :, :_N] = top_a.T
    x11_ref[0, :, _N:] = top_b.T


def kernel(x1, v1, v5, v6r, v7r):
    del v6r
    B, N, D = x1.shape

    x6 = pl.pallas_call(
        _colmax_body,
        grid=(B,),
        in_specs=[pl.BlockSpec((1, N, D), lambda b: (b, 0, 0))],
        out_specs=pl.BlockSpec((1, 1, D), lambda b: (b, 0, 0)),
        out_shape=jax.ShapeDtypeStruct((B, 1, D), jnp.float32),
    )(v5)

    strip = pl.BlockSpec((1, N, _TD), lambda b, d: (b, 0, d))
    x10, x11, x12 = pl.pallas_call(
        _main_body,
        grid=(B, D // _TD),
        in_specs=[
            strip,
            strip,
            strip,
            pl.BlockSpec((1, 1, N), lambda b, d: (b, 0, 0)),
        ],
        out_specs=[
            pl.BlockSpec((1, 2 * N, _TD), lambda b, d: (b, 0, d)),
            pl.BlockSpec((1, _TD, 2 * N), lambda b, d: (b, d, 0)),
            pl.BlockSpec((1, 2 * N, _TD), lambda b, d: (b, 0, d)),
        ],
        out_shape=[
            jax.ShapeDtypeStruct((B, 2 * N, D), jnp.float32),
            jax.ShapeDtypeStruct((B, D, 2 * N), jnp.float32),
            jax.ShapeDtypeStruct((B, 2 * N, D), jnp.float32),
        ],
    )(x1, v1, v7r, x6)
    return (x10, x11, x12)
